# Initial kernel scaffold; baseline (speedup 1.0000x reference)
import jax
import jax.numpy as jnp
from jax.experimental import pallas as pl


def kernel(input, scores):
    raise NotImplementedError("write your pallas kernel here")


# speedup vs baseline: 96.4942x; 96.4942x over previous
import functools

import jax
import jax.numpy as jnp
from jax.experimental import pallas as pl
from jax.experimental.pallas import tpu as pltpu

_MASK_ABS = 0x7FFFFFFF
_N_PIVOTS = 3
_N_ROUNDS = 20


def _count_body(pivots_ref, scores_ref, counts_ref):
    i = pl.program_id(0)

    @pl.when(i == 0)
    def _init():
        counts_ref[...] = jnp.zeros_like(counts_ref)

    bits = jax.lax.bitcast_convert_type(scores_ref[...], jnp.int32) & _MASK_ABS
    lane = jax.lax.broadcasted_iota(jnp.int32, (1, 128), 1)
    acc = jnp.zeros((1, 128), jnp.int32)
    for k in range(_N_PIVOTS):
        c = jnp.sum((bits < pivots_ref[k]).astype(jnp.int32))
        acc = acc + jnp.where(lane == k, c, 0)
    counts_ref[...] = counts_ref[...] + acc


def _make_count(rows, cols, block_rows):
    grid = rows // block_rows
    return pl.pallas_call(
        _count_body,
        grid=(grid,),
        in_specs=[
            pl.BlockSpec(memory_space=pltpu.SMEM),
            pl.BlockSpec((block_rows, cols), lambda i: (i, 0)),
        ],
        out_specs=pl.BlockSpec((1, 128), lambda i: (0, 0)),
        out_shape=jax.ShapeDtypeStruct((1, 128), jnp.int32),
    )


def _apply_body(t_ref, scores_ref, x_ref, out_ref):
    bits = jax.lax.bitcast_convert_type(scores_ref[...], jnp.int32) & _MASK_ABS
    mask = bits >= t_ref[0]
    x = x_ref[...]
    out_ref[...] = jnp.where(mask[None, :, :], jnp.maximum(x, 0.0), x)


def _make_apply(batch, rows, cols, block_rows):
    grid = rows // block_rows
    return pl.pallas_call(
        _apply_body,
        grid=(grid,),
        in_specs=[
            pl.BlockSpec(memory_space=pltpu.SMEM),
            pl.BlockSpec((block_rows, cols), lambda i: (i, 0)),
            pl.BlockSpec((batch, block_rows, cols), lambda i: (0, i, 0)),
        ],
        out_specs=pl.BlockSpec((batch, block_rows, cols), lambda i: (0, i, 0)),
        out_shape=jax.ShapeDtypeStruct((batch, rows, cols), jnp.float32),
    )


@jax.jit
def kernel(input, scores):
    batch, rows, cols = input.shape
    n = rows * cols
    j = n // 2
    block_rows = 256 if rows % 256 == 0 else rows

    count_fn = _make_count(rows, cols, block_rows)

    def round_fn(_, carry):
        lo, hi = carry
        q = jnp.maximum((hi - lo) // (_N_PIVOTS + 1), 1)
        pivots = lo + q * jnp.arange(1, _N_PIVOTS + 1, dtype=jnp.int32)
        counts = count_fn(pivots, scores)[0, : _N_PIVOTS]
        below = counts <= j
        lo = jnp.max(jnp.where(below, pivots, lo))
        hi = jnp.min(jnp.where(~below, pivots, hi))
        return lo, hi

    lo0 = jnp.int32(0)
    hi0 = jnp.int32(0x7F800001)
    t_bits, _ = jax.lax.fori_loop(0, _N_ROUNDS, round_fn, (lo0, hi0))

    apply_fn = _make_apply(batch, rows, cols, block_rows)
    return apply_fn(t_bits[None], scores, input)


# speedup vs baseline: 164.4177x; 1.7039x over previous
import numpy as np

import jax
import jax.numpy as jnp
from jax.experimental import pallas as pl
from jax.experimental.pallas import tpu as pltpu

_MASK_ABS = 0x7FFFFFFF
_HI_INIT = 0x7F800001
_N_ROUNDS = 11

_BOUND = float(np.sqrt(2.0 / 6.0) * np.sqrt(3.0 / 2048.0))
_DELTA = 8.6e-5
_SEED_PIVOTS = tuple(
    int(np.float32(v).view(np.int32))
    for v in (_BOUND / 2 - _DELTA, _BOUND / 2, _BOUND / 2 + _DELTA)
)


def _bits_of(scores):
    return jax.lax.bitcast_convert_type(scores, jnp.int32) & _MASK_ABS


def _dyn_pivots(lo, hi):
    q = jnp.maximum((hi - lo) // 4, 1)
    return [lo + q, lo + 2 * q, lo + 3 * q]


def _bracket_update(lo, hi, pivots, counts, j):
    for p, c in zip(pivots, counts):
        below = c <= j
        lo = jnp.where(below, jnp.maximum(lo, p), lo)
        hi = jnp.where(below, hi, jnp.minimum(hi, p))
    return lo, hi


def _thresh_body(scores_ref, lohi_ref, st_ref, cnt_ref, *, j, nblk):
    r = pl.program_id(0)
    b = pl.program_id(1)

    @pl.when((r == 0) & (b == 0))
    def _init():
        st_ref[0] = 0
        st_ref[1] = _HI_INIT

    @pl.when(b == 0)
    def _zero():
        for k in range(3):
            cnt_ref[k] = 0

    bits = _bits_of(scores_ref[...])
    lo = st_ref[0]
    hi = st_ref[1]

    @pl.when(r == 0)
    def _seed_count():
        for k, p in enumerate(_SEED_PIVOTS):
            cnt_ref[k] += jnp.sum((bits < p).astype(jnp.int32))

    @pl.when(r > 0)
    def _dyn_count():
        for k, p in enumerate(_dyn_pivots(lo, hi)):
            cnt_ref[k] += jnp.sum((bits < p).astype(jnp.int32))

    @pl.when(b == nblk - 1)
    def _update():
        pivots = [
            jnp.where(r == 0, jnp.int32(sp), dp)
            for sp, dp in zip(_SEED_PIVOTS, _dyn_pivots(lo, hi))
        ]
        counts = [cnt_ref[k] for k in range(3)]
        lo2, hi2 = _bracket_update(lo, hi, pivots, counts, j)
        st_ref[0] = lo2
        st_ref[1] = hi2

        @pl.when(r == _N_ROUNDS - 1)
        def _emit():
            lane = jax.lax.broadcasted_iota(jnp.int32, (1, 128), 1)
            lohi_ref[...] = jnp.where(lane == 0, lo2, jnp.where(lane == 1, hi2, 0))


def _count_body(pivots_ref, scores_ref, counts_ref):
    i = pl.program_id(0)

    @pl.when(i == 0)
    def _init():
        counts_ref[...] = jnp.zeros_like(counts_ref)

    bits = _bits_of(scores_ref[...])
    lane = jax.lax.broadcasted_iota(jnp.int32, (1, 128), 1)
    acc = jnp.zeros((1, 128), jnp.int32)
    for k in range(3):
        c = jnp.sum((bits < pivots_ref[k]).astype(jnp.int32))
        acc = acc + jnp.where(lane == k, c, 0)
    counts_ref[...] = counts_ref[...] + acc


def _apply_body(t_ref, scores_ref, x_ref, out_ref):
    mask = _bits_of(scores_ref[...]) >= t_ref[0]
    x = x_ref[...]
    out_ref[...] = jnp.where(mask[None, :, :], jnp.maximum(x, 0.0), x)


@jax.jit
def kernel(input, scores):
    import functools

    batch, rows, cols = input.shape
    n = rows * cols
    j = n // 2
    block_rows = 256 if rows % 256 == 0 else rows
    nblk = rows // block_rows

    lohi = pl.pallas_call(
        functools.partial(_thresh_body, j=j, nblk=nblk),
        grid=(_N_ROUNDS, nblk),
        in_specs=[pl.BlockSpec((block_rows, cols), lambda r, b: (b, 0))],
        out_specs=pl.BlockSpec((1, 128), lambda r, b: (0, 0)),
        out_shape=jax.ShapeDtypeStruct((1, 128), jnp.int32),
        scratch_shapes=[pltpu.SMEM((2,), jnp.int32), pltpu.SMEM((4,), jnp.int32)],
    )(scores)
    lo0, hi0 = lohi[0, 0], lohi[0, 1]

    count_fn = pl.pallas_call(
        _count_body,
        grid=(nblk,),
        in_specs=[
            pl.BlockSpec(memory_space=pltpu.SMEM),
            pl.BlockSpec((block_rows, cols), lambda i: (i, 0)),
        ],
        out_specs=pl.BlockSpec((1, 128), lambda i: (0, 0)),
        out_shape=jax.ShapeDtypeStruct((1, 128), jnp.int32),
    )

    def w_cond(carry):
        lo, hi = carry
        return hi - lo > 1

    def w_body(carry):
        lo, hi = carry
        pivots = jnp.stack(_dyn_pivots(lo, hi))
        counts = count_fn(pivots, scores)[0, :3]
        return _bracket_update(lo, hi, list(pivots), list(counts), j)

    t_bits, _ = jax.lax.while_loop(w_cond, w_body, (lo0, hi0))

    return pl.pallas_call(
        _apply_body,
        grid=(nblk,),
        in_specs=[
            pl.BlockSpec(memory_space=pltpu.SMEM),
            pl.BlockSpec((block_rows, cols), lambda i: (i, 0)),
            pl.BlockSpec((batch, block_rows, cols), lambda i: (0, i, 0)),
        ],
        out_specs=pl.BlockSpec((batch, block_rows, cols), lambda i: (0, i, 0)),
        out_shape=jax.ShapeDtypeStruct((batch, rows, cols), jnp.float32),
    )(t_bits[None], scores, input)
